# baseline (speedup 1.0000x reference)
import dataclasses
import functools

import jax
import jax.numpy as jnp
from jax import lax
from jax.experimental import pallas as pl
from jax.experimental.pallas import tpu as pltpu
from jax.experimental.pallas import tpu_sc as plsc

_N = 10000
_E = 320000
_C = 64
_K = 10
_EPS = 1e-5

_NP = _N + 16
_L = 16
_NTILES = 32
_CW = _C // _NTILES
_CH = 8000
_NCHUNK = _E // _CH


def _trunk_tc(x, W1, b1, W2, b2, W3, b3, s1, t1, s2, t2):
    def body(x_ref, w1_ref, b1_ref, w2_ref, b2_ref, w3_ref, b3_ref,
             s1_ref, t1_ref, s2_ref, t2_ref, o_ref):
        hi = lax.Precision.HIGHEST
        h = lax.dot_general(x_ref[...], w1_ref[...],
                            (((1,), (0,)), ((), ())), precision=hi)
        h = jnp.maximum((h + b1_ref[...]) * s1_ref[...] + t1_ref[...], 0.0)
        h = lax.dot_general(h, w2_ref[...],
                            (((1,), (0,)), ((), ())), precision=hi)
        h = jnp.maximum((h + b2_ref[...]) * s2_ref[...] + t2_ref[...], 0.0)
        h = lax.dot_general(h, w3_ref[...],
                            (((1,), (0,)), ((), ())), precision=hi)
        o_ref[...] = h + b3_ref[...]

    return pl.pallas_call(
        body,
        out_shape=jax.ShapeDtypeStruct((_N, _C), jnp.float32),
    )(x, W1, b1, W2, b2, W3, b3, s1, t1, s2, t2)


def _gcn_prop_sc(h0t, pe):
    mesh = plsc.VectorSubcoreMesh(core_axis_name="c", subcore_axis_name="s")
    cp = pltpu.CompilerParams()
    if "needs_layout_passes" in pltpu.CompilerParams.__dataclass_fields__:
        cp = dataclasses.replace(cp, needs_layout_passes=False)

    @functools.partial(
        pl.kernel,
        out_type=jax.ShapeDtypeStruct((_K + 1, _C, _N), jnp.float32),
        mesh=mesh,
        compiler_params=cp,
        scratch_types=[
            pltpu.VMEM((_NP,), jnp.float32),
            pltpu.VMEM((_N,), jnp.float32),
            pltpu.VMEM((_NP,), jnp.int32),
            pltpu.VMEM((_NP,), jnp.float32),
            pltpu.VMEM((_NP,), jnp.float32),
            pltpu.VMEM((_N,), jnp.float32),
            pltpu.VMEM((_N,), jnp.float32),
            pltpu.VMEM((_CH,), jnp.int32),
            pltpu.VMEM((_CH,), jnp.int32),
            pltpu.SemaphoreType.DMA,
            pltpu.SemaphoreType.DMA,
        ],
    )
    def k(h0_hbm, pe_hbm, out_hbm, dis_v, dis2_v, gp_v, a0_v, a1_v,
          h0v, h1v, ebuf0, ebuf1, sem0, sem1):
        t = lax.axis_index("c") * 16 + lax.axis_index("s")
        c0 = t * _CW
        ebufs = (ebuf0, ebuf1)
        sems = (sem0, sem1)
        accs = (a0_v, a1_v)
        hbs = (h0v, h1v)
        ones = jnp.ones((_L,), jnp.float32)
        m16 = jnp.full((_L,), 0xFFFF, jnp.int32)
        sh16 = jnp.full((_L,), 16, jnp.int32)
        mhi = jnp.full((_L,), -65536, jnp.int32)
        h8 = jnp.full((_L,), 0x8000, jnp.int32)

        def pack_g(gn0, gn1):
            b0 = lax.bitcast_convert_type(gn0, jnp.int32)
            b1 = lax.bitcast_convert_type(gn1, jnp.int32)
            lo = lax.shift_right_logical(b0 + h8, sh16)
            hi_ = jnp.bitwise_and(b1 + h8, mhi)
            return jnp.bitwise_or(lo, hi_)

        def stream_edges(proc):
            pltpu.async_copy(pe_hbm.at[pl.ds(0, _CH)], ebuf0, sem0)

            @pl.loop(0, _NCHUNK, step=2)
            def _chunks(ci):
                for b in range(2):
                    cur = ci + b
                    nxt = cur + 1

                    @pl.when(nxt < _NCHUNK)
                    def _start():
                        pltpu.async_copy(pe_hbm.at[pl.ds(nxt * _CH, _CH)],
                                         ebufs[1 - b], sems[1 - b])

                    pltpu.make_async_copy(pe_hbm.at[pl.ds(cur * _CH, _CH)],
                                          ebufs[b], sems[b]).wait()
                    buf = ebufs[b]

                    @plsc.parallel_loop(0, _CH // _L, unroll=6)
                    def _windows(w):
                        proc(buf[pl.ds(w * _L, _L)])

        @plsc.parallel_loop(0, _N // _L, unroll=5)
        def _init_deg(i):
            dis_v[pl.ds(i * _L, _L)] = ones

        def deg_proc(p):
            cdst = lax.shift_right_logical(p, sh16)
            plsc.addupdate_scatter(dis_v, [cdst], ones)

        stream_edges(deg_proc)

        magic = jnp.full((_L,), 0x5F3759DF, jnp.int32)
        one_i = jnp.full((_L,), 1, jnp.int32)
        half = jnp.full((_L,), 0.5, jnp.float32)
        th = jnp.full((_L,), 1.5, jnp.float32)

        @plsc.parallel_loop(0, _N // _L, unroll=5)
        def _rsqrt(i):
            s = pl.ds(i * _L, _L)
            d = dis_v[s]
            yi = magic - lax.shift_right_arithmetic(
                lax.bitcast_convert_type(d, jnp.int32), one_i)
            y = lax.bitcast_convert_type(yi, jnp.float32)
            for _ in range(4):
                y = y * (th - half * d * y * y)
            dis_v[s] = y
            dis2_v[s] = y * y

        for j in range(_CW):
            pltpu.sync_copy(h0_hbm.at[c0 + j], hbs[j])
            pltpu.sync_copy(hbs[j], out_hbm.at[0].at[c0 + j])

        @plsc.parallel_loop(0, _N // _L, unroll=5)
        def _init_g(i):
            s = pl.ds(i * _L, _L)
            gv0 = dis_v[s] * hbs[0][s]
            gv1 = dis_v[s] * hbs[1][s]
            accs[0][s] = gv0
            accs[1][s] = gv1
            gp_v[s] = pack_g(gv0, gv1)

        def prop_proc(p):
            r = jnp.bitwise_and(p, m16)
            cdst = lax.shift_right_logical(p, sh16)
            v = plsc.load_gather(gp_v, [r])
            v0 = lax.bitcast_convert_type(lax.shift_left(v, sh16), jnp.float32)
            v1 = lax.bitcast_convert_type(jnp.bitwise_and(v, mhi), jnp.float32)
            plsc.addupdate_scatter(accs[0], [cdst], v0)
            plsc.addupdate_scatter(accs[1], [cdst], v1)

        for step in range(_K):
            stream_edges(prop_proc)

            @plsc.parallel_loop(0, _N // _L, unroll=5)
            def _scale(i):
                s = pl.ds(i * _L, _L)
                a0 = accs[0][s]
                a1 = accs[1][s]
                hbs[0][s] = dis_v[s] * a0
                hbs[1][s] = dis_v[s] * a1
                gn0 = dis2_v[s] * a0
                gn1 = dis2_v[s] * a1
                accs[0][s] = gn0
                accs[1][s] = gn1
                gp_v[s] = pack_g(gn0, gn1)

            for j in range(_CW):
                pltpu.sync_copy(hbs[j], out_hbm.at[step + 1].at[c0 + j])

    return k(h0t, pe)


def _combine_tc(preds, proj_w, proj_b):
    def body(p_ref, w_ref, b_ref, o_ref):
        w = w_ref[...]
        b = b_ref[0, 0]
        comb = jnp.zeros((_C, _N), jnp.float32)
        for kk in range(_K + 1):
            pk = p_ref[kk]
            sk = jnp.sum(pk * w, axis=0, keepdims=True)
            rsk = jax.nn.sigmoid(sk + b)
            comb = comb + rsk * pk
        m = jnp.max(comb, axis=0, keepdims=True)
        z = comb - m
        lse = jnp.log(jnp.sum(jnp.exp(z), axis=0, keepdims=True))
        o_ref[...] = z - lse

    return pl.pallas_call(
        body,
        out_shape=jax.ShapeDtypeStruct((_C, _N), jnp.float32),
    )(preds, proj_w, proj_b)


def kernel(x, edge_index, W1, b1, W2, b2, W3, b3, bn1_w, bn1_b, bn1_mean,
           bn1_var, bn2_w, bn2_b, bn2_mean, bn2_var, proj_w, proj_b):
    s1 = (bn1_w / jnp.sqrt(bn1_var + _EPS)).reshape(1, -1)
    t1 = (bn1_b - bn1_mean * bn1_w / jnp.sqrt(bn1_var + _EPS)).reshape(1, -1)
    s2 = (bn2_w / jnp.sqrt(bn2_var + _EPS)).reshape(1, -1)
    t2 = (bn2_b - bn2_mean * bn2_w / jnp.sqrt(bn2_var + _EPS)).reshape(1, -1)

    h0 = _trunk_tc(x, W1, b1.reshape(1, -1), W2, b2.reshape(1, -1),
                   W3, b3.reshape(1, -1), s1, t1, s2, t2)

    row = edge_index[0].astype(jnp.int32)
    col = edge_index[1].astype(jnp.int32)
    pe = jnp.where(row == col, jnp.int32((_N << 16) | _N),
                   jnp.bitwise_or(jnp.left_shift(col, 16), row))

    preds = _gcn_prop_sc(h0.T, pe)
    outt = _combine_tc(preds, proj_w, proj_b.reshape(1, 1))
    return outt.T


# speedup vs baseline: 1.0185x; 1.0185x over previous
import dataclasses
import functools

import jax
import jax.numpy as jnp
from jax import lax
from jax.experimental import pallas as pl
from jax.experimental.pallas import tpu as pltpu
from jax.experimental.pallas import tpu_sc as plsc

_N = 10000
_E = 320000
_C = 64
_K = 10
_EPS = 1e-5

_NP = _N + 16
_L = 16
_NTILES = 32
_CW = _C // _NTILES
_CH = 8000
_NCHUNK = _E // _CH


def _trunk_tc(x, W1, b1, W2, b2, W3, b3, s1, t1, s2, t2):
    def body(x_ref, w1_ref, b1_ref, w2_ref, b2_ref, w3_ref, b3_ref,
             s1_ref, t1_ref, s2_ref, t2_ref, o_ref):
        hi = lax.Precision.HIGHEST
        h = lax.dot_general(x_ref[...], w1_ref[...],
                            (((1,), (0,)), ((), ())), precision=hi)
        h = jnp.maximum((h + b1_ref[...]) * s1_ref[...] + t1_ref[...], 0.0)
        h = lax.dot_general(h, w2_ref[...],
                            (((1,), (0,)), ((), ())), precision=hi)
        h = jnp.maximum((h + b2_ref[...]) * s2_ref[...] + t2_ref[...], 0.0)
        h = lax.dot_general(h, w3_ref[...],
                            (((1,), (0,)), ((), ())), precision=hi)
        o_ref[...] = h + b3_ref[...]

    return pl.pallas_call(
        body,
        out_shape=jax.ShapeDtypeStruct((_N, _C), jnp.float32),
    )(x, W1, b1, W2, b2, W3, b3, s1, t1, s2, t2)


def _gcn_prop_sc(h0t, pe):
    mesh = plsc.VectorSubcoreMesh(core_axis_name="c", subcore_axis_name="s")
    cp = pltpu.CompilerParams()
    if "needs_layout_passes" in pltpu.CompilerParams.__dataclass_fields__:
        cp = dataclasses.replace(cp, needs_layout_passes=False)

    @functools.partial(
        pl.kernel,
        out_type=jax.ShapeDtypeStruct((_K + 1, _C, _N), jnp.float32),
        mesh=mesh,
        compiler_params=cp,
        scratch_types=[
            pltpu.VMEM((_NP,), jnp.float32),
            pltpu.VMEM((_N,), jnp.float32),
            pltpu.VMEM((_NP,), jnp.int32),
            pltpu.VMEM((_NP,), jnp.float32),
            pltpu.VMEM((_NP,), jnp.float32),
            pltpu.VMEM((_N,), jnp.float32),
            pltpu.VMEM((_N,), jnp.float32),
            pltpu.VMEM((_CH,), jnp.int32),
            pltpu.VMEM((_CH,), jnp.int32),
            pltpu.SemaphoreType.DMA,
            pltpu.SemaphoreType.DMA,
        ],
    )
    def k(h0_hbm, pe_hbm, out_hbm, dis_v, dis2_v, gp_v, a0_v, a1_v,
          h0v, h1v, ebuf0, ebuf1, sem0, sem1):
        t = lax.axis_index("c") * 16 + lax.axis_index("s")
        c0 = t * _CW
        ebufs = (ebuf0, ebuf1)
        sems = (sem0, sem1)
        accs = (a0_v, a1_v)
        hbs = (h0v, h1v)
        ones = jnp.ones((_L,), jnp.float32)
        m16 = jnp.full((_L,), 0xFFFF, jnp.int32)
        sh16 = jnp.full((_L,), 16, jnp.int32)
        mhi = jnp.full((_L,), -65536, jnp.int32)
        h8 = jnp.full((_L,), 0x8000, jnp.int32)

        def pack_g(gn0, gn1):
            b0 = lax.bitcast_convert_type(gn0, jnp.int32)
            b1 = lax.bitcast_convert_type(gn1, jnp.int32)
            lo = lax.shift_right_logical(b0 + h8, sh16)
            hi_ = jnp.bitwise_and(b1 + h8, mhi)
            return jnp.bitwise_or(lo, hi_)

        def stream_edges(proc):
            pltpu.async_copy(pe_hbm.at[pl.ds(0, _CH)], ebuf0, sem0)

            @pl.loop(0, _NCHUNK, step=2)
            def _chunks(ci):
                for b in range(2):
                    cur = ci + b
                    nxt = cur + 1

                    @pl.when(nxt < _NCHUNK)
                    def _start():
                        pltpu.async_copy(pe_hbm.at[pl.ds(nxt * _CH, _CH)],
                                         ebufs[1 - b], sems[1 - b])

                    pltpu.make_async_copy(pe_hbm.at[pl.ds(cur * _CH, _CH)],
                                          ebufs[b], sems[b]).wait()
                    buf = ebufs[b]

                    @plsc.parallel_loop(0, _CH // _L, unroll=4)
                    def _windows(w):
                        proc(buf[pl.ds(w * _L, _L)])

        @plsc.parallel_loop(0, _N // _L, unroll=5)
        def _init_deg(i):
            dis_v[pl.ds(i * _L, _L)] = ones

        def deg_proc(p):
            cdst = lax.shift_right_logical(p, sh16)
            plsc.addupdate_scatter(dis_v, [cdst], ones)

        stream_edges(deg_proc)

        magic = jnp.full((_L,), 0x5F3759DF, jnp.int32)
        one_i = jnp.full((_L,), 1, jnp.int32)
        half = jnp.full((_L,), 0.5, jnp.float32)
        th = jnp.full((_L,), 1.5, jnp.float32)

        @plsc.parallel_loop(0, _N // _L, unroll=5)
        def _rsqrt(i):
            s = pl.ds(i * _L, _L)
            d = dis_v[s]
            yi = magic - lax.shift_right_arithmetic(
                lax.bitcast_convert_type(d, jnp.int32), one_i)
            y = lax.bitcast_convert_type(yi, jnp.float32)
            for _ in range(4):
                y = y * (th - half * d * y * y)
            dis_v[s] = y
            dis2_v[s] = y * y

        for j in range(_CW):
            pltpu.sync_copy(h0_hbm.at[c0 + j], hbs[j])
            pltpu.sync_copy(hbs[j], out_hbm.at[0].at[c0 + j])

        @plsc.parallel_loop(0, _N // _L, unroll=5)
        def _init_g(i):
            s = pl.ds(i * _L, _L)
            gv0 = dis_v[s] * hbs[0][s]
            gv1 = dis_v[s] * hbs[1][s]
            accs[0][s] = gv0
            accs[1][s] = gv1
            gp_v[s] = pack_g(gv0, gv1)

        def prop_proc(p):
            r = jnp.bitwise_and(p, m16)
            cdst = lax.shift_right_logical(p, sh16)
            v = plsc.load_gather(gp_v, [r])
            v0 = lax.bitcast_convert_type(lax.shift_left(v, sh16), jnp.float32)
            v1 = lax.bitcast_convert_type(jnp.bitwise_and(v, mhi), jnp.float32)
            plsc.addupdate_scatter(accs[0], [cdst], v0)
            plsc.addupdate_scatter(accs[1], [cdst], v1)

        for step in range(_K):
            stream_edges(prop_proc)

            @plsc.parallel_loop(0, _N // _L, unroll=5)
            def _scale(i):
                s = pl.ds(i * _L, _L)
                a0 = accs[0][s]
                a1 = accs[1][s]
                hbs[0][s] = dis_v[s] * a0
                hbs[1][s] = dis_v[s] * a1
                gn0 = dis2_v[s] * a0
                gn1 = dis2_v[s] * a1
                accs[0][s] = gn0
                accs[1][s] = gn1
                gp_v[s] = pack_g(gn0, gn1)

            for j in range(_CW):
                pltpu.sync_copy(hbs[j], out_hbm.at[step + 1].at[c0 + j])

    return k(h0t, pe)


def _combine_tc(preds, proj_w, proj_b):
    def body(p_ref, w_ref, b_ref, o_ref):
        w = w_ref[...]
        b = b_ref[0, 0]
        comb = jnp.zeros((_C, _N), jnp.float32)
        for kk in range(_K + 1):
            pk = p_ref[kk]
            sk = jnp.sum(pk * w, axis=0, keepdims=True)
            rsk = jax.nn.sigmoid(sk + b)
            comb = comb + rsk * pk
        m = jnp.max(comb, axis=0, keepdims=True)
        z = comb - m
        lse = jnp.log(jnp.sum(jnp.exp(z), axis=0, keepdims=True))
        o_ref[...] = z - lse

    return pl.pallas_call(
        body,
        out_shape=jax.ShapeDtypeStruct((_C, _N), jnp.float32),
    )(preds, proj_w, proj_b)


def kernel(x, edge_index, W1, b1, W2, b2, W3, b3, bn1_w, bn1_b, bn1_mean,
           bn1_var, bn2_w, bn2_b, bn2_mean, bn2_var, proj_w, proj_b):
    s1 = (bn1_w / jnp.sqrt(bn1_var + _EPS)).reshape(1, -1)
    t1 = (bn1_b - bn1_mean * bn1_w / jnp.sqrt(bn1_var + _EPS)).reshape(1, -1)
    s2 = (bn2_w / jnp.sqrt(bn2_var + _EPS)).reshape(1, -1)
    t2 = (bn2_b - bn2_mean * bn2_w / jnp.sqrt(bn2_var + _EPS)).reshape(1, -1)

    h0 = _trunk_tc(x, W1, b1.reshape(1, -1), W2, b2.reshape(1, -1),
                   W3, b3.reshape(1, -1), s1, t1, s2, t2)

    row = edge_index[0].astype(jnp.int32)
    col = edge_index[1].astype(jnp.int32)
    pe = jnp.where(row == col, jnp.int32((_N << 16) | _N),
                   jnp.bitwise_or(jnp.left_shift(col, 16), row))

    preds = _gcn_prop_sc(h0.T, pe)
    outt = _combine_tc(preds, proj_w, proj_b.reshape(1, 1))
    return outt.T
